# baseline (speedup 1.0000x reference)
import functools

import jax
import jax.numpy as jnp
from jax import lax
from jax.experimental import pallas as pl
from jax.experimental.pallas import tpu as pltpu
from jax.experimental.pallas import tpu_sc as plsc

BATCH = 4096
SEQ = 200
EMBED = 64
VOCAB = 1000000

NUM_CORES = 2
NUM_SUBCORES = 16
NUM_WORKERS = NUM_CORES * NUM_SUBCORES
BPW = BATCH // NUM_WORKERS
SBLK = SEQ // 8


@functools.partial(
    pl.kernel,
    out_type=jax.ShapeDtypeStruct((SEQ, EMBED // 8, NUM_WORKERS, 8 * BPW), jnp.float32),
    mesh=plsc.VectorSubcoreMesh(core_axis_name="c", subcore_axis_name="s"),
    compiler_params=pltpu.CompilerParams(
        use_tc_tiling_on_sc=False, needs_layout_passes=False),
    scratch_types=[
        pltpu.VMEM((SBLK, 8, BPW), jnp.int32),
        pltpu.VMEM((2, BPW, EMBED), jnp.float32),
        pltpu.VMEM((2, 8, 8 * BPW), jnp.float32),
        pltpu.VMEM((SEQ, EMBED), jnp.float32),
        pltpu.SemaphoreType.DMA,
        pltpu.SemaphoreType.DMA,
        pltpu.SemaphoreType.DMA,
        pltpu.SemaphoreType.DMA,
    ],
)
def _sc_embed(x_hbm, tok_hbm, pos_hbm, out_hbm,
              idx_v, rows_v, til_v, pos_v, g0, g1, o0, o1):
    wid = lax.axis_index("s") * NUM_CORES + lax.axis_index("c")

    pltpu.sync_copy(pos_hbm, pos_v)
    pltpu.sync_copy(x_hbm.at[:, wid], idx_v)

    def scale_flat(i, carry):
        blk = i // 64
        row = (i // 8) % 8
        col = (i % 8) * 16
        v = idx_v[blk, row, pl.ds(col, 16)]
        idx_v[blk, row, pl.ds(col, 16)] = v + v
        return carry
    lax.fori_loop(0, SBLK * 8 * 8, scale_flat, 0)

    def gather(s, p):
        gsem = (g0, g1)[p]
        blk = s // 8
        row = s % 8
        return pltpu.async_copy(
            tok_hbm.at[idx_v.at[blk, row]], rows_v.at[p], gsem)

    def wait_gather(p):
        gsem = (g0, g1)[p]
        pltpu.make_async_copy(
            tok_hbm.at[idx_v.at[0, 0]], rows_v.at[p], gsem).wait()

    def store(s, p):
        osem = (o0, o1)[p]
        return pltpu.async_copy(til_v.at[p], out_hbm.at[s, :, wid], osem)

    def wait_store(s, p):
        osem = (o0, o1)[p]
        pltpu.make_async_copy(til_v.at[p], out_hbm.at[s, :, wid], osem).wait()

    gather(0, 0)

    def s_body(step, carry):
        for p in range(2):
            s = step * 2 + p

            @pl.when(s + 1 < SEQ)
            def _():
                gather(s + 1, 1 - p)

            wait_gather(p)

            @pl.when(s >= 2)
            def _():
                wait_store(s, p)

            def tok(t, tcarry):
                iota = lax.iota(jnp.int32, 16)
                tv = jnp.full((16,), t, jnp.int32)
                for c in range(4):
                    e = iota + 16 * c
                    v = (rows_v[p, t, pl.ds(16 * c, 16)]
                         + pos_v[s, pl.ds(16 * c, 16)])
                    d0 = lax.shift_right_logical(e, 3)
                    d1 = lax.shift_left(jnp.bitwise_and(e, 7), 7) + tv
                    plsc.store_scatter(til_v.at[p], [d0, d1], v)
                return tcarry
            lax.fori_loop(0, BPW, tok, 0)

            store(s, p)
        return carry

    lax.fori_loop(0, SEQ // 2, s_body, 0)
    wait_store(SEQ - 2, 0)
    wait_store(SEQ - 1, 1)


def kernel(x, token_table, position_table):
    xq = (
        jnp.transpose(x.astype(jnp.int32))
        .reshape(SBLK, 8, NUM_WORKERS, BPW)
        .transpose(0, 2, 1, 3)
    )
    tp = jnp.pad(token_table, ((0, 0), (0, 128 - EMBED))).reshape(2 * VOCAB, EMBED)
    out6 = _sc_embed(xq, tp, position_table)
    return (
        out6.reshape(SEQ, 8, NUM_WORKERS, 8, BPW)
        .transpose(2, 4, 0, 1, 3)
        .reshape(BATCH, SEQ, EMBED)
    )
